# baseline (speedup 1.0000x reference)
import functools

import jax
import jax.numpy as jnp
from jax import lax
from jax.experimental import pallas as pl
from jax.experimental.pallas import tpu as pltpu
from jax.experimental.pallas import tpu_sc as plsc

NUM_FIELDS = 26
VOCAB = 100000
EMBED_DIM = 32
BATCH = 4096

_INFO = plsc.get_sparse_core_info()
_NC = _INFO.num_cores
_NS = _INFO.num_subcores
_NW = _NC * _NS
_TOTAL = BATCH * NUM_FIELDS
_PER_W = _TOTAL // _NW
_LANES = 16
_ROW_W = 128
_PACK = _ROW_W // EMBED_DIM
_WIDE_ROWS = NUM_FIELDS * VOCAB // _PACK
_OUT_WIDE = _TOTAL // _PACK
_CHUNK = 256
_NCHUNK = _PER_W // _CHUNK
_OW_PER_CHUNK = _CHUNK // _PACK


def _make_sc_gather():
    mesh = plsc.VectorSubcoreMesh(core_axis_name="c", subcore_axis_name="s")

    @functools.partial(
        pl.kernel,
        mesh=mesh,
        out_type=jax.ShapeDtypeStruct((_OUT_WIDE, _ROW_W), jnp.float32),
        scratch_types=[
            pltpu.VMEM((_PER_W,), jnp.int32),
            pltpu.VMEM((_PER_W,), jnp.int32),
            pltpu.VMEM((2, _CHUNK, _ROW_W), jnp.float32),
            pltpu.VMEM((_OW_PER_CHUNK, _ROW_W), jnp.float32),
            pltpu.SemaphoreType.DMA,
        ],
    )
    def k(idx_hbm, tab_hbm, out_hbm, idx_g, qoff, gbuf, obuf, sem):
        wid = lax.axis_index("s") * _NC + lax.axis_index("c")
        base = wid * _PER_W

        pltpu.sync_copy(idx_hbm.at[pl.ds(base, _PER_W)], idx_g)

        lane = lax.iota(jnp.int32, _LANES)

        def prep(t, _):
            pos = lane + t * _LANES
            r = (pos % NUM_FIELDS) * VOCAB + idx_g[pl.ds(t * _LANES, _LANES)]
            sl = pl.ds(t * _LANES, _LANES)
            idx_g[sl] = lax.shift_right_logical(r, 2)
            qoff[sl] = lax.shift_left(lax.bitwise_and(r, 3), 5)
            return 0

        lax.fori_loop(0, _PER_W // _LANES, prep, 0)

        def fire(k_, buf):
            d0 = pltpu.async_copy(
                tab_hbm.at[idx_g.at[pl.ds(k_ * _CHUNK, 128)]],
                gbuf.at[buf, pl.ds(0, 128)], sem)
            d1 = pltpu.async_copy(
                tab_hbm.at[idx_g.at[pl.ds(k_ * _CHUNK + 128, 128)]],
                gbuf.at[buf, pl.ds(128, 128)], sem)
            return d0, d1

        def select_and_flush(k_, buf):
            qbase = k_ * _CHUNK

            def sel(t, _):
                qv = qoff[pl.ds(qbase + t * _LANES, _LANES)]
                r0 = t * _LANES
                for j in range(_LANES):
                    qo = qv[j]
                    ow = t * 4 + (j // 4)
                    col = (j % 4) * EMBED_DIM
                    obuf[ow, pl.ds(col, _LANES)] = (
                        gbuf[buf, r0 + j, pl.ds(qo, _LANES)])
                    obuf[ow, pl.ds(col + _LANES, _LANES)] = (
                        gbuf[buf, r0 + j, pl.ds(qo + _LANES, _LANES)])
                return 0

            lax.fori_loop(0, _CHUNK // _LANES, sel, 0)
            pltpu.sync_copy(
                obuf,
                out_hbm.at[pl.ds(wid * (_PER_W // _PACK) + k_ * _OW_PER_CHUNK,
                                 _OW_PER_CHUNK)])

        descs = fire(0, 0)
        for k_ in range(1, _NCHUNK + 1):
            nxt = None
            if k_ < _NCHUNK:
                nxt = fire(k_, k_ % 2)
            descs[0].wait()
            descs[1].wait()
            select_and_flush(k_ - 1, (k_ - 1) % 2)
            descs = nxt

    return k


_sc_gather = _make_sc_gather()


@jax.jit
def kernel(indices, tables):
    idx_flat = indices.astype(jnp.int32).reshape(_TOTAL)
    tab_wide = tables.reshape(_WIDE_ROWS, _ROW_W)
    out = _sc_gather(idx_flat, tab_wide)
    return out.reshape(BATCH, NUM_FIELDS, EMBED_DIM)
